# baseline (speedup 1.0000x reference)
import jax
import jax.numpy as jnp
from jax.experimental import pallas as pl
from jax.experimental.pallas import tpu as pltpu

_BB = 64


def _add_kernel(x_ref, e_ref, o_ref):
    ld = x_ref.shape[1]
    o_ref[...] = x_ref[...] + e_ref[:, :ld]


def kernel(x, encoding):
    B, L, D = x.shape
    xf = x.reshape(B, L * D)
    ef = encoding.reshape(1, encoding.shape[0] * D)
    grid = (B // _BB,)
    out = pl.pallas_call(
        _add_kernel,
        grid=grid,
        in_specs=[
            pl.BlockSpec((_BB, L * D), lambda i: (i, 0)),
            pl.BlockSpec((1, ef.shape[1]), lambda i: (0, 0)),
        ],
        out_specs=pl.BlockSpec((_BB, L * D), lambda i: (i, 0)),
        out_shape=jax.ShapeDtypeStruct((B, L * D), x.dtype),
        compiler_params=pltpu.CompilerParams(
            dimension_semantics=("parallel",),
        ),
    )(xf, ef)
    return out.reshape(B, L, D)


# speedup vs baseline: 1.0058x; 1.0058x over previous
import jax
import jax.numpy as jnp
from jax.experimental import pallas as pl
from jax.experimental.pallas import tpu as pltpu

_BB = 32
_K = 8


def _make_body(num_chunks, ld):
    def body(x_hbm, e_vmem, o_hbm, in_buf, out_buf, in_sem, out_sem):
        e = e_vmem[:, :ld]

        def start_in(i):
            s = i % _K
            pltpu.make_async_copy(
                x_hbm.at[pl.ds(i * _BB, _BB), :], in_buf.at[s], in_sem.at[s]
            ).start()

        for i in range(min(_K, num_chunks)):
            start_in(i)
        for i in range(num_chunks):
            s = i % _K
            pltpu.make_async_copy(
                x_hbm.at[pl.ds(i * _BB, _BB), :], in_buf.at[s], in_sem.at[s]
            ).wait()
            if i >= _K:
                pltpu.make_async_copy(
                    out_buf.at[s],
                    o_hbm.at[pl.ds((i - _K) * _BB, _BB), :],
                    out_sem.at[s],
                ).wait()
            out_buf[s] = in_buf[s] + e
            pltpu.make_async_copy(
                out_buf.at[s], o_hbm.at[pl.ds(i * _BB, _BB), :], out_sem.at[s]
            ).start()
            if i + _K < num_chunks:
                start_in(i + _K)
        for i in range(max(0, num_chunks - _K), num_chunks):
            s = i % _K
            pltpu.make_async_copy(
                out_buf.at[s], o_hbm.at[pl.ds(i * _BB, _BB), :], out_sem.at[s]
            ).wait()

    return body


def kernel(x, encoding):
    B, L, D = x.shape
    LD = L * D
    num_chunks = B // _BB
    xf = x.reshape(B, LD)
    ef = encoding.reshape(1, encoding.shape[0] * D)
    out = pl.pallas_call(
        _make_body(num_chunks, LD),
        in_specs=[
            pl.BlockSpec(memory_space=pl.ANY),
            pl.BlockSpec(memory_space=pltpu.VMEM),
        ],
        out_specs=pl.BlockSpec(memory_space=pl.ANY),
        out_shape=jax.ShapeDtypeStruct((B, LD), x.dtype),
        scratch_shapes=[
            pltpu.VMEM((_K, _BB, LD), x.dtype),
            pltpu.VMEM((_K, _BB, LD), x.dtype),
            pltpu.SemaphoreType.DMA((_K,)),
            pltpu.SemaphoreType.DMA((_K,)),
        ],
    )(xf, ef)
    return out.reshape(B, L, D)
